# baseline (speedup 1.0000x reference)
import jax
import jax.numpy as jnp
from jax.experimental import pallas as pl
from jax.experimental.pallas import tpu as pltpu

NUM_ROLES = 100
NUM_TOKENS = 50
EMBED_DIM = 4096
BT = 56
BR = 8


def _select_body(rid_ref, x_ref, o_ref):
    r8 = rid_ref[0] % BR
    o_ref[...] = x_ref[:, r8, :]


def kernel(embeds, role_id):
    x = jnp.transpose(embeds, (1, 0, 2))
    rid = jnp.asarray(role_id, jnp.int32).reshape(1)
    grid = (NUM_TOKENS + BT - 1) // BT
    return pl.pallas_call(
        _select_body,
        grid_spec=pltpu.PrefetchScalarGridSpec(
            num_scalar_prefetch=1,
            grid=(grid,),
            in_specs=[
                pl.BlockSpec(
                    (BT, BR, EMBED_DIM),
                    lambda i, rid_ref: (i, rid_ref[0] // BR, 0),
                ),
            ],
            out_specs=pl.BlockSpec((BT, EMBED_DIM), lambda i, rid_ref: (i, 0)),
        ),
        out_shape=jax.ShapeDtypeStruct((NUM_TOKENS, EMBED_DIM), jnp.float32),
    )(rid, x)


# speedup vs baseline: 1.0027x; 1.0027x over previous
import jax
import jax.numpy as jnp
from jax.experimental import pallas as pl
from jax.experimental.pallas import tpu as pltpu

NUM_ROLES = 100
NUM_TOKENS = 50
EMBED_DIM = 4096
BT = 24
BR = 8


def _select_body(rid_ref, x_ref, o_ref):
    r8 = rid_ref[0] % BR
    o_ref[...] = x_ref[:, r8, :]


def kernel(embeds, role_id):
    x = jnp.transpose(embeds, (1, 0, 2))
    rid = jnp.asarray(role_id, jnp.int32).reshape(1)
    grid = (NUM_TOKENS + BT - 1) // BT
    return pl.pallas_call(
        _select_body,
        grid_spec=pltpu.PrefetchScalarGridSpec(
            num_scalar_prefetch=1,
            grid=(grid,),
            in_specs=[
                pl.BlockSpec(
                    (BT, BR, EMBED_DIM),
                    lambda i, rid_ref: (i, rid_ref[0] // BR, 0),
                ),
            ],
            out_specs=pl.BlockSpec((BT, EMBED_DIM), lambda i, rid_ref: (i, 0)),
        ),
        out_shape=jax.ShapeDtypeStruct((NUM_TOKENS, EMBED_DIM), jnp.float32),
    )(rid, x)


# speedup vs baseline: 1.0547x; 1.0519x over previous
import jax
import jax.numpy as jnp
from jax.experimental import pallas as pl
from jax.experimental.pallas import tpu as pltpu

NUM_ROLES = 100
NUM_TOKENS = 50
EMBED_DIM = 4096
BT = 40
BR = 8


def _select_body(rid_ref, x_ref, o_ref):
    r8 = rid_ref[0] % BR
    o_ref[...] = x_ref[:, r8, :]


def kernel(embeds, role_id):
    x = jnp.transpose(embeds, (1, 0, 2))
    rid = jnp.asarray(role_id, jnp.int32).reshape(1)
    grid = (NUM_TOKENS + BT - 1) // BT
    return pl.pallas_call(
        _select_body,
        grid_spec=pltpu.PrefetchScalarGridSpec(
            num_scalar_prefetch=1,
            grid=(grid,),
            in_specs=[
                pl.BlockSpec(
                    (BT, BR, EMBED_DIM),
                    lambda i, rid_ref: (i, rid_ref[0] // BR, 0),
                ),
            ],
            out_specs=pl.BlockSpec((BT, EMBED_DIM), lambda i, rid_ref: (i, 0)),
        ),
        out_shape=jax.ShapeDtypeStruct((NUM_TOKENS, EMBED_DIM), jnp.float32),
    )(rid, x)
